# Initial kernel scaffold; baseline (speedup 1.0000x reference)
import jax
import jax.numpy as jnp
from jax.experimental import pallas as pl


def kernel(input):
    raise NotImplementedError("write your pallas kernel here")


# speedup vs baseline: 36.7787x; 36.7787x over previous
import math

import jax
import jax.numpy as jnp
from jax import lax
from jax.experimental import pallas as pl


_RATIO = 0.1
_DEMOTE = 1.0


def _body(x_ref, o_ref):
    n = x_ref.shape[1]
    k = math.ceil(_RATIO * n)
    v = jnp.maximum(x_ref[...], 0.0)
    bits = lax.bitcast_convert_type(v, jnp.int32)

    rows = x_ref.shape[0]
    lo0 = jnp.zeros((rows, 1), jnp.int32)
    hi0 = jnp.full((rows, 1), 0x7F800000, jnp.int32)

    def step(_, carry):
        lo, hi = carry
        mid = lo + (hi - lo) // 2
        cnt = jnp.sum((bits >= mid).astype(jnp.int32), axis=1, keepdims=True)
        ge = cnt >= k
        return jnp.where(ge, mid, lo), jnp.where(ge, hi, mid)

    lo, hi = lax.fori_loop(0, 31, step, (lo0, hi0))
    t_bits = lo
    t = lax.bitcast_convert_type(t_bits, jnp.float32)

    gt = bits > t_bits
    s = jnp.sum(jnp.where(gt, v, 0.0), axis=1, keepdims=True)
    c = jnp.sum(gt.astype(jnp.float32), axis=1, keepdims=True)
    tot = jnp.sum(v, axis=1, keepdims=True)
    topk = s + t * (k - c)
    row = topk / k - _DEMOTE * (tot - topk) / (n - k)
    o_ref[...] = jnp.mean(row).reshape(1, 1)


def kernel(input):
    out = pl.pallas_call(
        _body,
        out_shape=jax.ShapeDtypeStruct((1, 1), jnp.float32),
    )(input)
    return out[0, 0]
